# baseline (speedup 1.0000x reference)
import functools

import jax
import jax.numpy as jnp
from jax import lax
from jax.experimental import pallas as pl
from jax.experimental.pallas import tpu as pltpu
from jax.experimental.pallas import tpu_sc as plsc

_S, _E, _NQ = 512, 256, 128
_G = 64
_NW = 32
_GPW = _S // _NW
_L = 16


def _sc_build_body(edges_hbm, a_hbm, edges_v, deg_v, dis_v, a_v, sem):
    f32, i32 = jnp.float32, jnp.int32
    wid = lax.axis_index("s") * 2 + lax.axis_index("c")

    def zero_row(i, c):
        def zero_chunk(j, c2):
            a_v[i, pl.ds(j * _L, _L)] = jnp.zeros((_L,), f32)
            return c2
        return lax.fori_loop(0, _NQ // _L, zero_chunk, c)
    lax.fori_loop(0, _NQ, zero_row, 0)

    def per_graph(gi, c):
        g = wid * _GPW + gi
        pltpu.sync_copy(edges_hbm.at[g], edges_v)

        def deg_init(j, c2):
            deg_v[pl.ds(j * _L, _L)] = jnp.ones((_L,), f32)
            return c2
        lax.fori_loop(0, _NQ // _L, deg_init, c)

        def deg_count(ch, c2):
            dst = edges_v[1, pl.ds(ch * _L, _L)]
            plsc.addupdate_scatter(deg_v, [dst], jnp.ones((_L,), f32))
            return c2
        lax.fori_loop(0, _E // _L, deg_count, c)

        def rsqrt_chunk(j, c2):
            d = deg_v[pl.ds(j * _L, _L)]
            yi = jnp.int32(0x5F3759DF) - (plsc.bitcast(d, i32) >> 1)
            y = plsc.bitcast(yi, f32)
            y = y * (1.5 - 0.5 * d * y * y)
            y = y * (1.5 - 0.5 * d * y * y)
            y = y * (1.5 - 0.5 * d * y * y)
            dis_v[pl.ds(j * _L, _L)] = y
            return c2
        lax.fori_loop(0, _NQ // _L, rsqrt_chunk, c)

        def edge_scatter(sign):
            def go(ch, c2):
                src = edges_v[0, pl.ds(ch * _L, _L)]
                dst = edges_v[1, pl.ds(ch * _L, _L)]
                v = plsc.load_gather(dis_v, [src]) * plsc.load_gather(dis_v, [dst])
                plsc.addupdate_scatter(a_v, [dst, src], sign * v)
                return c2
            return go

        def diag_scatter(sign):
            def go(j, c2):
                idx = lax.iota(i32, _L) + j * _L
                dv = dis_v[pl.ds(j * _L, _L)]
                plsc.addupdate_scatter(a_v, [idx, idx], sign * (dv * dv))
                return c2
            return go

        lax.fori_loop(0, _E // _L, edge_scatter(1.0), c)
        lax.fori_loop(0, _NQ // _L, diag_scatter(1.0), c)
        pltpu.sync_copy(a_v, a_hbm.at[g])
        lax.fori_loop(0, _E // _L, edge_scatter(-1.0), c)
        lax.fori_loop(0, _NQ // _L, diag_scatter(-1.0), c)
        return c
    lax.fori_loop(0, _GPW, per_graph, 0)


_sc_build = functools.partial(
    pl.kernel,
    mesh=plsc.VectorSubcoreMesh(core_axis_name="c", subcore_axis_name="s"),
    compiler_params=pltpu.CompilerParams(needs_layout_passes=False),
    out_type=jax.ShapeDtypeStruct((_S, _NQ, _NQ), jnp.float32),
    scratch_types=[
        pltpu.VMEM((2, _E), jnp.int32),
        pltpu.VMEM((_NQ,), jnp.float32),
        pltpu.VMEM((_NQ,), jnp.float32),
        pltpu.VMEM((_NQ, _NQ), jnp.float32),
        pltpu.SemaphoreType.DMA,
    ],
)(_sc_build_body)


def _gnn_body(a_ref, qe_ref, w0_ref, b0_ref, w1_ref, b1_ref,
              w2_ref, b2_ref, out_ref):
    f32 = jnp.float32
    G = a_ref.shape[0]
    h0 = jnp.dot(qe_ref[...], w0_ref[...], preferred_element_type=f32)

    a_big = a_ref[...].reshape(G * _NQ, _NQ)
    x = jnp.maximum(jnp.dot(a_big, h0, preferred_element_type=f32)
                    + b0_ref[...], 0.0)

    h = jnp.dot(x, w1_ref[...], preferred_element_type=f32)
    x = jnp.concatenate(
        [jnp.maximum(jnp.dot(a_big[g * _NQ:(g + 1) * _NQ],
                             h[g * _NQ:(g + 1) * _NQ],
                             preferred_element_type=f32) + b1_ref[...], 0.0)
         for g in range(G)], axis=0)

    h = jnp.dot(x, w2_ref[...], preferred_element_type=f32)
    for g in range(G):
        out_ref[g * _NQ:(g + 1) * _NQ, :] = jnp.maximum(
            jnp.dot(a_big[g * _NQ:(g + 1) * _NQ], h[g * _NQ:(g + 1) * _NQ],
                    preferred_element_type=f32) + b2_ref[...], 0.0)


def kernel(slice_matrices, qubit_embeddings, W0, b0, W1, b1, W2, b2):
    edges = slice_matrices.astype(jnp.int32)
    a_all = _sc_build(edges)
    d0, d1 = W0.shape
    d2 = W1.shape[1]
    d3 = W2.shape[1]
    return pl.pallas_call(
        _gnn_body,
        grid=(_S // _G,),
        in_specs=[
            pl.BlockSpec((_G, _NQ, _NQ), lambda i: (i, 0, 0)),
            pl.BlockSpec((_NQ, d0), lambda i: (0, 0)),
            pl.BlockSpec((d0, d1), lambda i: (0, 0)),
            pl.BlockSpec((1, d1), lambda i: (0, 0)),
            pl.BlockSpec((d1, d2), lambda i: (0, 0)),
            pl.BlockSpec((1, d2), lambda i: (0, 0)),
            pl.BlockSpec((d2, d3), lambda i: (0, 0)),
            pl.BlockSpec((1, d3), lambda i: (0, 0)),
        ],
        out_specs=pl.BlockSpec((_G * _NQ, d3), lambda i: (i, 0)),
        out_shape=jax.ShapeDtypeStruct((_S * _NQ, d3), jnp.float32),
    )(a_all, qubit_embeddings, W0, b0.reshape(1, -1), W1, b1.reshape(1, -1),
      W2, b2.reshape(1, -1))


# speedup vs baseline: 1.1132x; 1.1132x over previous
import functools

import jax
import jax.numpy as jnp
from jax import lax
from jax.experimental import pallas as pl
from jax.experimental.pallas import tpu as pltpu
from jax.experimental.pallas import tpu_sc as plsc

_S, _E, _NQ = 512, 256, 128
_G = 64
_NW = 32
_GPW = _S // _NW
_L = 16


def _sc_build_body(edges_hbm, a_hbm, edges_v, deg_v, dis_v, a_v, sem0, sem1):
    f32, i32 = jnp.float32, jnp.int32
    wid = lax.axis_index("s") * 2 + lax.axis_index("c")
    sems = (sem0, sem1)

    def zero_slot(b):
        def zero_row(i, c):
            def zero_chunk(j, c2):
                a_v[b, i, pl.ds(j * _L, _L)] = jnp.zeros((_L,), f32)
                return c2
            return lax.fori_loop(0, _NQ // _L, zero_chunk, c)
        lax.fori_loop(0, _NQ, zero_row, 0)
    zero_slot(0)
    zero_slot(1)

    pltpu.sync_copy(edges_hbm.at[pl.ds(wid * _GPW, _GPW)], edges_v)

    def edge_pass(b, gi, sign):
        def go(ch, c2):
            src = edges_v[gi, 0, pl.ds(ch * _L, _L)]
            dst = edges_v[gi, 1, pl.ds(ch * _L, _L)]
            v = (plsc.load_gather(dis_v.at[b], [src])
                 * plsc.load_gather(dis_v.at[b], [dst]))
            plsc.addupdate_scatter(a_v.at[b], [dst, src], sign * v)
            return c2
        lax.fori_loop(0, _E // _L, go, 0)

        def go_diag(j, c2):
            idx = lax.iota(i32, _L) + j * _L
            dv = dis_v[b, pl.ds(j * _L, _L)]
            plsc.addupdate_scatter(a_v.at[b], [idx, idx], sign * (dv * dv))
            return c2
        lax.fori_loop(0, _NQ // _L, go_diag, 0)

    def build(b, gi):
        g = wid * _GPW + gi

        def deg_init(j, c2):
            deg_v[pl.ds(j * _L, _L)] = jnp.ones((_L,), f32)
            return c2
        lax.fori_loop(0, _NQ // _L, deg_init, 0)

        def deg_count(ch, c2):
            dst = edges_v[gi, 1, pl.ds(ch * _L, _L)]
            plsc.addupdate_scatter(deg_v, [dst], jnp.ones((_L,), f32))
            return c2
        lax.fori_loop(0, _E // _L, deg_count, 0)

        def rsqrt_chunk(j, c2):
            d = deg_v[pl.ds(j * _L, _L)]
            yi = jnp.int32(0x5F3759DF) - (plsc.bitcast(d, i32) >> 1)
            y = plsc.bitcast(yi, f32)
            y = y * (1.5 - 0.5 * d * y * y)
            y = y * (1.5 - 0.5 * d * y * y)
            y = y * (1.5 - 0.5 * d * y * y)
            dis_v[b, pl.ds(j * _L, _L)] = y
            return c2
        lax.fori_loop(0, _NQ // _L, rsqrt_chunk, 0)

        edge_pass(b, gi, 1.0)
        pltpu.make_async_copy(a_v.at[b], a_hbm.at[g], sems[b]).start()

    def pair(p, c):
        for b in (0, 1):
            gi = 2 * p + b

            @pl.when(p > 0)
            def _drain():
                g_old = wid * _GPW + gi - 2
                pltpu.make_async_copy(a_v.at[b], a_hbm.at[g_old],
                                      sems[b]).wait()
                edge_pass(b, gi - 2, -1.0)

            build(b, gi)
        return c
    lax.fori_loop(0, _GPW // 2, pair, 0)

    for b in (0, 1):
        g_last = wid * _GPW + _GPW - 2 + b
        pltpu.make_async_copy(a_v.at[b], a_hbm.at[g_last], sems[b]).wait()


_sc_build = functools.partial(
    pl.kernel,
    mesh=plsc.VectorSubcoreMesh(core_axis_name="c", subcore_axis_name="s"),
    compiler_params=pltpu.CompilerParams(needs_layout_passes=False),
    out_type=jax.ShapeDtypeStruct((_S, _NQ, _NQ), jnp.float32),
    scratch_types=[
        pltpu.VMEM((_GPW, 2, _E), jnp.int32),
        pltpu.VMEM((_NQ,), jnp.float32),
        pltpu.VMEM((2, _NQ), jnp.float32),
        pltpu.VMEM((2, _NQ, _NQ), jnp.float32),
        pltpu.SemaphoreType.DMA,
        pltpu.SemaphoreType.DMA,
    ],
)(_sc_build_body)


def _gnn_body(a_ref, qe_ref, w0_ref, b0_ref, w1_ref, b1_ref,
              w2_ref, b2_ref, out_ref):
    f32 = jnp.float32
    G = a_ref.shape[0]
    h0 = jnp.dot(qe_ref[...], w0_ref[...], preferred_element_type=f32)

    a_big = a_ref[...].reshape(G * _NQ, _NQ)
    x = jnp.maximum(jnp.dot(a_big, h0, preferred_element_type=f32)
                    + b0_ref[...], 0.0)

    h = jnp.dot(x, w1_ref[...], preferred_element_type=f32)
    x = jnp.concatenate(
        [jnp.maximum(jnp.dot(a_big[g * _NQ:(g + 1) * _NQ],
                             h[g * _NQ:(g + 1) * _NQ],
                             preferred_element_type=f32) + b1_ref[...], 0.0)
         for g in range(G)], axis=0)

    h = jnp.dot(x, w2_ref[...], preferred_element_type=f32)
    for g in range(G):
        out_ref[g * _NQ:(g + 1) * _NQ, :] = jnp.maximum(
            jnp.dot(a_big[g * _NQ:(g + 1) * _NQ], h[g * _NQ:(g + 1) * _NQ],
                    preferred_element_type=f32) + b2_ref[...], 0.0)


def kernel(slice_matrices, qubit_embeddings, W0, b0, W1, b1, W2, b2):
    edges = slice_matrices.astype(jnp.int32)
    a_all = _sc_build(edges)
    d0, d1 = W0.shape
    d2 = W1.shape[1]
    d3 = W2.shape[1]
    return pl.pallas_call(
        _gnn_body,
        grid=(_S // _G,),
        in_specs=[
            pl.BlockSpec((_G, _NQ, _NQ), lambda i: (i, 0, 0)),
            pl.BlockSpec((_NQ, d0), lambda i: (0, 0)),
            pl.BlockSpec((d0, d1), lambda i: (0, 0)),
            pl.BlockSpec((1, d1), lambda i: (0, 0)),
            pl.BlockSpec((d1, d2), lambda i: (0, 0)),
            pl.BlockSpec((1, d2), lambda i: (0, 0)),
            pl.BlockSpec((d2, d3), lambda i: (0, 0)),
            pl.BlockSpec((1, d3), lambda i: (0, 0)),
        ],
        out_specs=pl.BlockSpec((_G * _NQ, d3), lambda i: (i, 0)),
        out_shape=jax.ShapeDtypeStruct((_S * _NQ, d3), jnp.float32),
    )(a_all, qubit_embeddings, W0, b0.reshape(1, -1), W1, b1.reshape(1, -1),
      W2, b2.reshape(1, -1))


# speedup vs baseline: 1.2945x; 1.1629x over previous
import jax
import jax.numpy as jnp
from jax.experimental import pallas as pl
from jax.experimental.pallas import tpu as pltpu

_S, _E, _NQ = 512, 256, 128
_G = 64


def _gnn_body(edges_ref, qe_ref, w0_ref, b0_ref, w1_ref, b1_ref,
              w2_ref, b2_ref, out_ref):
    f32 = jnp.float32
    G = edges_ref.shape[0]
    h0 = jnp.dot(qe_ref[...], w0_ref[...], preferred_element_type=f32)

    row = jax.lax.broadcasted_iota(jnp.int32, (_NQ, _E), 0)
    rr = jax.lax.broadcasted_iota(jnp.int32, (_NQ, _NQ), 0)
    cc = jax.lax.broadcasted_iota(jnp.int32, (_NQ, _NQ), 1)
    eye = (rr == cc).astype(f32)

    dims_ee = (((1,), (1,)), ((), ()))
    As = []
    for g in range(G):
        src = edges_ref[g, 0:1, :]
        dst = edges_ref[g, 1:2, :]
        oh_src = (src == row).astype(f32)
        oh_dst = (dst == row).astype(f32)
        deg = jnp.sum(oh_dst, axis=1, keepdims=True) + 1.0
        dis = jax.lax.rsqrt(deg)
        a = jax.lax.dot_general(oh_dst * dis, oh_src * dis, dims_ee,
                                preferred_element_type=f32)
        As.append(a + eye * (dis * dis))

    a_big = jnp.concatenate(As, axis=0)
    x = jnp.maximum(jnp.dot(a_big, h0, preferred_element_type=f32)
                    + b0_ref[...], 0.0)

    h = jnp.dot(x, w1_ref[...], preferred_element_type=f32)
    x = jnp.concatenate(
        [jnp.maximum(jnp.dot(a_big[g * _NQ:(g + 1) * _NQ],
                             h[g * _NQ:(g + 1) * _NQ],
                             preferred_element_type=f32) + b1_ref[...], 0.0)
         for g in range(G)], axis=0)

    h = jnp.dot(x, w2_ref[...], preferred_element_type=f32)
    for g in range(G):
        out_ref[g * _NQ:(g + 1) * _NQ, :] = jnp.maximum(
            jnp.dot(a_big[g * _NQ:(g + 1) * _NQ], h[g * _NQ:(g + 1) * _NQ],
                    preferred_element_type=f32) + b2_ref[...], 0.0)


def kernel(slice_matrices, qubit_embeddings, W0, b0, W1, b1, W2, b2):
    edges = slice_matrices.astype(jnp.int32)
    d0, d1 = W0.shape
    d2 = W1.shape[1]
    d3 = W2.shape[1]
    return pl.pallas_call(
        _gnn_body,
        grid=(_S // _G,),
        in_specs=[
            pl.BlockSpec((_G, 2, _E), lambda i: (i, 0, 0)),
            pl.BlockSpec((_NQ, d0), lambda i: (0, 0)),
            pl.BlockSpec((d0, d1), lambda i: (0, 0)),
            pl.BlockSpec((1, d1), lambda i: (0, 0)),
            pl.BlockSpec((d1, d2), lambda i: (0, 0)),
            pl.BlockSpec((1, d2), lambda i: (0, 0)),
            pl.BlockSpec((d2, d3), lambda i: (0, 0)),
            pl.BlockSpec((1, d3), lambda i: (0, 0)),
        ],
        out_specs=pl.BlockSpec((_G * _NQ, d3), lambda i: (i, 0)),
        out_shape=jax.ShapeDtypeStruct((_S * _NQ, d3), jnp.float32),
        compiler_params=pltpu.CompilerParams(
            dimension_semantics=("parallel",)),
    )(edges, qubit_embeddings, W0, b0.reshape(1, -1), W1, b1.reshape(1, -1),
      W2, b2.reshape(1, -1))
